# baseline (speedup 1.0000x reference)
import functools

import jax
import jax.numpy as jnp
import numpy as np
from jax import lax
from jax.experimental import pallas as pl
from jax.experimental.pallas import tpu as pltpu
from jax.experimental.pallas import tpu_sc as plsc

B, C, H, W = 64, 1, 32, 4096
N_CHUNKS = 16
CHUNK = W // N_CHUNKS

HBAND = 8
N_BANDS = H // HBAND
TASKS = B * N_BANDS
NUM_WORKERS = 32
TASKS_PER_WORKER = TASKS // NUM_WORKERS

NBUF = 3
DEPTH = 2


def _perm_table() -> np.ndarray:
    base = jax.random.key(42)
    perms = jax.jit(
        jax.vmap(lambda b: jax.random.permutation(jax.random.fold_in(base, b),
                                                  N_CHUNKS))
    )(jnp.arange(B))
    return np.asarray(jax.device_get(perms)).astype(np.int32)


_PERMS = _perm_table()


def _sc_shuffle(x, ptbl):
    mesh = plsc.VectorSubcoreMesh(core_axis_name="c", subcore_axis_name="s")

    @functools.partial(
        pl.kernel,
        mesh=mesh,
        out_type=jax.ShapeDtypeStruct((1, C, H, B * W), jnp.float32),
        scratch_types=[
            pltpu.VMEM((B * N_CHUNKS,), jnp.int32),
        ]
        + [pltpu.VMEM((HBAND, W), jnp.float32)] * NBUF
        + [pltpu.SemaphoreType.DMA] * (2 * NBUF),
    )
    def k(x_hbm, ptbl_hbm, out_hbm, ptbl_v, b0, b1, b2,
          g0, g1, g2, s0, s1, s2):
        bufs = (b0, b1, b2)
        gsem = (g0, g1, g2)
        ssem = (s0, s1, s2)
        wid = lax.axis_index("c") * 16 + lax.axis_index("s")
        pltpu.sync_copy(ptbl_hbm, ptbl_v)

        def gather(t):
            task = wid * TASKS_PER_WORKER + t
            b = task // N_BANDS
            band = task % N_BANDS
            buf = bufs[t % NBUF]
            sem = gsem[t % NBUF]
            row = ptbl_v[pl.ds(b * N_CHUNKS, N_CHUNKS)]
            descs = []
            for j in range(N_CHUNKS):
                p = row[j]
                descs.append(pltpu.async_copy(
                    x_hbm.at[b, 0, pl.ds(band * HBAND, HBAND),
                             pl.ds(p * CHUNK, CHUNK)],
                    buf.at[:, pl.ds(j * CHUNK, CHUNK)],
                    sem))
            del descs
            return [pltpu.make_async_copy(
                x_hbm.at[b, 0, pl.ds(band * HBAND, HBAND), :], buf, sem)]

        def store(t):
            task = wid * TASKS_PER_WORKER + t
            b = task // N_BANDS
            band = task % N_BANDS
            return pltpu.async_copy(
                bufs[t % NBUF],
                out_hbm.at[0, 0, pl.ds(band * HBAND, HBAND),
                           pl.ds(b * W, W)],
                ssem[t % NBUF])

        gd = {t: gather(t) for t in range(DEPTH)}
        sd = {}
        for t in range(TASKS_PER_WORKER):
            for d in gd[t]:
                d.wait()
            sd[t] = store(t)
            u = t + DEPTH
            if u < TASKS_PER_WORKER:
                prev = u - NBUF
                if prev >= 0:
                    sd[prev].wait()
                gd[u] = gather(u)
        for t in range(TASKS_PER_WORKER - NBUF, TASKS_PER_WORKER):
            sd[t].wait()

    return k(x, ptbl)


def kernel(input_batch):
    return _sc_shuffle(input_batch, jnp.asarray(_PERMS.reshape(-1)))


# speedup vs baseline: 1.0304x; 1.0304x over previous
import functools

import jax
import jax.numpy as jnp
import numpy as np
from jax import lax
from jax.experimental import pallas as pl
from jax.experimental.pallas import tpu as pltpu
from jax.experimental.pallas import tpu_sc as plsc

B, C, H, W = 64, 1, 32, 4096
N_CHUNKS = 16
CHUNK = W // N_CHUNKS

HBAND = 8
N_BANDS = H // HBAND
TASKS = B * N_BANDS
NUM_WORKERS = 32
TASKS_PER_WORKER = TASKS // NUM_WORKERS

NBUF = 3
DEPTH = 2


def _perm_table() -> np.ndarray:
    base = jax.random.key(42)
    perms = jax.jit(
        jax.vmap(lambda b: jax.random.permutation(jax.random.fold_in(base, b),
                                                  N_CHUNKS))
    )(jnp.arange(B))
    return np.asarray(jax.device_get(perms)).astype(np.int32)


_PERMS = _perm_table()


def _sc_shuffle(x, ptbl):
    mesh = plsc.VectorSubcoreMesh(core_axis_name="c", subcore_axis_name="s")

    @functools.partial(
        pl.kernel,
        mesh=mesh,
        out_type=jax.ShapeDtypeStruct((1, C, H, B * W), jnp.float32),
        scratch_types=[
            pltpu.VMEM((B * N_CHUNKS,), jnp.int32),
        ]
        + [pltpu.VMEM((HBAND, W), jnp.float32)] * NBUF
        + [pltpu.SemaphoreType.DMA] * (2 * NBUF),
    )
    def k(x_hbm, ptbl_hbm, out_hbm, ptbl_v, b0, b1, b2,
          g0, g1, g2, s0, s1, s2):
        bufs = (b0, b1, b2)
        gsem = (g0, g1, g2)
        ssem = (s0, s1, s2)
        wid = lax.axis_index("c") * 16 + lax.axis_index("s")
        pltpu.sync_copy(ptbl_hbm, ptbl_v)

        def gather(t):
            task = wid * TASKS_PER_WORKER + t
            b = task // N_BANDS
            band = task % N_BANDS
            buf = bufs[t % NBUF]
            sem = gsem[t % NBUF]
            row = ptbl_v[pl.ds(b * N_CHUNKS, N_CHUNKS)]
            descs = []
            for j in range(N_CHUNKS):
                p = row[j]
                descs.append(pltpu.async_copy(
                    x_hbm.at[b, 0, pl.ds(band * HBAND, HBAND),
                             pl.ds(p * CHUNK, CHUNK)],
                    buf.at[:, pl.ds(j * CHUNK, CHUNK)],
                    sem))
            return descs

        def store(t):
            task = wid * TASKS_PER_WORKER + t
            b = task // N_BANDS
            band = task % N_BANDS
            return pltpu.async_copy(
                bufs[t % NBUF],
                out_hbm.at[0, 0, pl.ds(band * HBAND, HBAND),
                           pl.ds(b * W, W)],
                ssem[t % NBUF])

        gd = {t: gather(t) for t in range(DEPTH)}
        sd = {}
        for t in range(TASKS_PER_WORKER):
            for d in gd[t]:
                d.wait()
            sd[t] = store(t)
            u = t + DEPTH
            if u < TASKS_PER_WORKER:
                prev = u - NBUF
                if prev >= 0:
                    sd[prev].wait()
                gd[u] = gather(u)
        for t in range(TASKS_PER_WORKER - NBUF, TASKS_PER_WORKER):
            sd[t].wait()

    return k(x, ptbl)


def kernel(input_batch):
    return _sc_shuffle(input_batch, jnp.asarray(_PERMS.reshape(-1)))
